# baseline (speedup 1.0000x reference)
import functools

import jax
import jax.numpy as jnp
from jax import lax
from jax.experimental import pallas as pl
from jax.experimental.pallas import tpu as pltpu
from jax.experimental.pallas import tpu_sc as plsc

N_NODES = 10000
N_PAD = 10240
E_EDGES = 320000
NW = 32
CHUNK = 128
K_CHUNKS = 80
E_PAD = NW * K_CHUNKS * CHUNK
TILE_ROWS = N_PAD // 16

BLK = 512
D1 = 128
D2 = 48
DEG1 = 117
DEG2 = 42
OUT_D = 24

_PREC = lax.Precision.DEFAULT
OUT_BLK = 400
OUT_GRID = N_NODES // OUT_BLK


def _mm2_body(x_ref, wl_ref, wr_ref, b_ref, t_ref, s_ref, *, ones_col):
    x = x_ref[...]
    t = jnp.dot(x, wl_ref[...], preferred_element_type=jnp.float32,
                precision=_PREC)
    col = lax.broadcasted_iota(jnp.int32, t.shape, 1)
    t_ref[...] = jnp.where(col == ones_col, 1.0, t)
    s_ref[...] = jnp.dot(x, wr_ref[...], preferred_element_type=jnp.float32,
                         precision=_PREC) + b_ref[...]


def _mid_body(a0_ref, a1_ref, s_ref, wl_ref, wr_ref, b_ref, t_ref, s2_ref,
              *, deg_col, ones_col):
    asum = a0_ref[0] + a1_ref[0]
    deg = asum[:, deg_col:deg_col + 1]
    inv = 1.0 / jnp.maximum(deg, 1.0)
    h = jnp.maximum(asum * inv + s_ref[...], 0.0)
    t = jnp.dot(h, wl_ref[...], preferred_element_type=jnp.float32,
                precision=_PREC)
    col = lax.broadcasted_iota(jnp.int32, t.shape, 1)
    t_ref[...] = jnp.where(col == ones_col, 1.0, t)
    s2_ref[...] = jnp.dot(h, wr_ref[...], preferred_element_type=jnp.float32,
                          precision=_PREC) + b_ref[...]


def _h2_body(a0_ref, a1_ref, s_ref, h_ref, *, deg_col):
    asum = a0_ref[0] + a1_ref[0]
    deg = asum[:, deg_col:deg_col + 1]
    inv = 1.0 / jnp.maximum(deg, 1.0)
    h = jnp.maximum(asum * inv + s_ref[...], 0.0)
    col = lax.broadcasted_iota(jnp.int32, h.shape, 1)
    h_ref[...] = jnp.where(col == deg_col, 1.0, h)


def _out_body(m0_ref, m1_ref, h_ref, wml_ref, wmr_ref, bm_ref,
              wvl_ref, wvr_ref, bv_ref, mu_ref, lv_ref, *, deg_col):
    msum = m0_ref[0] + m1_ref[0]
    deg = msum[:, deg_col:deg_col + 1]
    mean = msum * (1.0 / jnp.maximum(deg, 1.0))
    h = h_ref[...]
    mu_ref[...] = (jnp.dot(mean, wml_ref[...], preferred_element_type=jnp.float32, precision=_PREC)
                   + jnp.dot(h, wmr_ref[...], preferred_element_type=jnp.float32, precision=_PREC)
                   + bm_ref[...])
    lv_ref[...] = (jnp.dot(mean, wvl_ref[...], preferred_element_type=jnp.float32, precision=_PREC)
                   + jnp.dot(h, wvr_ref[...], preferred_element_type=jnp.float32, precision=_PREC)
                   + bv_ref[...])


def _row_blocks(d):
    return pl.BlockSpec((BLK, d), lambda i: (i, 0))


def _full(shape):
    ndim = len(shape)
    return pl.BlockSpec(shape, lambda i: (0,) * ndim)


def _agg_part(part, d):
    return pl.BlockSpec((1, BLK, d), lambda i, _p=part: (_p, i, 0))


_GRID = N_PAD // BLK


def _mm2(xp, wl, wr, b, d_out, ones_col):
    return pl.pallas_call(
        functools.partial(_mm2_body, ones_col=ones_col),
        grid=(_GRID,),
        in_specs=[_row_blocks(128), _full(wl.shape), _full(wr.shape), _full(b.shape)],
        out_specs=[_row_blocks(d_out), _row_blocks(d_out)],
        out_shape=[jax.ShapeDtypeStruct((N_PAD, d_out), jnp.float32)] * 2,
    )(xp, wl, wr, b)


def _mid(agg, s1, wl, wr, b, d_in, d_out, deg_col, ones_col):
    return pl.pallas_call(
        functools.partial(_mid_body, deg_col=deg_col, ones_col=ones_col),
        grid=(_GRID,),
        in_specs=[_agg_part(0, d_in), _agg_part(1, d_in), _row_blocks(d_in),
                  _full(wl.shape), _full(wr.shape), _full(b.shape)],
        out_specs=[_row_blocks(d_out), _row_blocks(d_out)],
        out_shape=[jax.ShapeDtypeStruct((N_PAD, d_out), jnp.float32)] * 2,
    )(agg, agg, s1, wl, wr, b)


def _h2(agg, s2, d, deg_col):
    return pl.pallas_call(
        functools.partial(_h2_body, deg_col=deg_col),
        grid=(_GRID,),
        in_specs=[_agg_part(0, d), _agg_part(1, d), _row_blocks(d)],
        out_specs=_row_blocks(d),
        out_shape=jax.ShapeDtypeStruct((N_PAD, d), jnp.float32),
    )(agg, agg, s2)


def _heads(agg, h2, wml, wmr, bm, wvl, wvr, bv, d, deg_col):
    rb = pl.BlockSpec((OUT_BLK, d), lambda i: (i, 0))
    ab0 = pl.BlockSpec((1, OUT_BLK, d), lambda i: (0, i, 0))
    ab1 = pl.BlockSpec((1, OUT_BLK, d), lambda i: (1, i, 0))
    ob = pl.BlockSpec((OUT_BLK, OUT_D), lambda i: (i, 0))
    return pl.pallas_call(
        functools.partial(_out_body, deg_col=deg_col),
        grid=(OUT_GRID,),
        in_specs=[ab0, ab1, rb,
                  _full(wml.shape), _full(wmr.shape), _full(bm.shape),
                  _full(wvl.shape), _full(wvr.shape), _full(bv.shape)],
        out_specs=[ob, ob],
        out_shape=[jax.ShapeDtypeStruct((N_NODES, OUT_D), jnp.float32)] * 2,
    )(agg, agg, h2, wml, wmr, bm, wvl, wvr, bv)


KA = 128
KB = K_CHUNKS * 2 - KA
N_STAGES = 4
STAGE_MAX = max(KA, KB) // N_STAGES


def _segsum_body(t_hbm, src_hbm, dst_hbm, out_hbm,
                 src_v, dst_v, rows_v, acc, sem0, sem1, *, d):
    c = lax.axis_index("c")
    s = lax.axis_index("s")

    zv = jnp.zeros((16,), jnp.float32)

    def zrow(i, carry):
        for k2 in range(d // 16):
            rows_v[0, i, pl.ds(k2 * 16, 16)] = zv
        return carry

    lax.fori_loop(0, CHUNK, zrow, 0)
    base = s * TILE_ROWS
    for r in range(TILE_ROWS // CHUNK):
        pltpu.sync_copy(rows_v.at[0], acc.at[pl.ds(base + r * CHUNK, CHUNK)])
    plsc.subcore_barrier()

    def gather(j, buf, sem):
        pltpu.async_copy(t_hbm.at[src_v.at[j]], rows_v.at[buf], sem)

    def gwait(j, buf, sem):
        pltpu.make_async_copy(t_hbm.at[src_v.at[j]], rows_v.at[buf], sem).wait()

    def run(chunk_base, k):
        stage = k // N_STAGES
        pairs = stage // 2
        for h in range(N_STAGES):
            off = chunk_base + h * stage
            pltpu.sync_copy(src_hbm.at[pl.ds(off, stage)],
                            src_v.at[pl.ds(0, stage)])
            pltpu.sync_copy(dst_hbm.at[pl.ds(off, stage)],
                            dst_v.at[pl.ds(0, stage)])
            gather(0, 0, sem0)

            def body(jj, carry):
                j0 = 2 * jj
                gwait(j0, 0, sem0)
                gather(j0 + 1, 1, sem1)
                pltpu.sync_copy(rows_v.at[0], acc.at[dst_v.at[j0]], add=True)

                @pl.when(jj < pairs - 1)
                def _():
                    gather(j0 + 2, 0, sem0)

                gwait(j0 + 1, 1, sem1)
                pltpu.sync_copy(rows_v.at[1], acc.at[dst_v.at[j0 + 1]],
                                add=True)
                return carry

            lax.fori_loop(0, pairs, body, 0)

    @pl.when(c == 0)
    def _():
        run(s * KA, KA)

    if KB > 0:
        @pl.when(c == 1)
        def _():
            run(16 * KA + s * KB, KB)

    plsc.subcore_barrier()
    pltpu.sync_copy(acc.at[pl.ds(base, TILE_ROWS)],
                    out_hbm.at[c, pl.ds(base, TILE_ROWS)])


NBUF = 4


def _segsum_deep_body(t_hbm, src_hbm, dst_hbm, out_hbm,
                      src_v, dst_v, rows_v, acc, sem0, sem1, sem2, sem3,
                      *, d):
    c = lax.axis_index("c")
    s = lax.axis_index("s")
    sems = (sem0, sem1, sem2, sem3)

    zv = jnp.zeros((16,), jnp.float32)

    def zrow(i, carry):
        for k2 in range(d // 16):
            rows_v[0, i, pl.ds(k2 * 16, 16)] = zv
        return carry

    lax.fori_loop(0, CHUNK, zrow, 0)
    base = s * TILE_ROWS
    for r in range(TILE_ROWS // CHUNK):
        pltpu.sync_copy(rows_v.at[0], acc.at[pl.ds(base + r * CHUNK, CHUNK)])
    plsc.subcore_barrier()

    def gather(j, b):
        pltpu.async_copy(t_hbm.at[src_v.at[j]], rows_v.at[b], sems[b])

    def gwait(j, b):
        pltpu.make_async_copy(t_hbm.at[src_v.at[j]], rows_v.at[b],
                              sems[b]).wait()

    def run(chunk_base, k):
        pltpu.sync_copy(src_hbm.at[pl.ds(chunk_base, k)],
                        src_v.at[pl.ds(0, k)])
        pltpu.sync_copy(dst_hbm.at[pl.ds(chunk_base, k)],
                        dst_v.at[pl.ds(0, k)])
        for b in range(NBUF - 1):
            gather(b, b)

        def body(g, carry):
            j0 = g * NBUF
            for b in range(NBUF):
                j = j0 + b
                nxt = j + NBUF - 1
                nb = (b + NBUF - 1) % NBUF

                @pl.when(nxt < k)
                def _():
                    gather(nxt, nb)

                gwait(j, b)
                pltpu.sync_copy(rows_v.at[b], acc.at[dst_v.at[j]], add=True)
            return carry

        lax.fori_loop(0, k // NBUF, body, 0)

    @pl.when(c == 0)
    def _():
        run(s * KA, KA)

    if KB > 0:
        @pl.when(c == 1)
        def _():
            run(16 * KA + s * KB, KB)

    plsc.subcore_barrier()
    pltpu.sync_copy(acc.at[pl.ds(base, TILE_ROWS)],
                    out_hbm.at[c, pl.ds(base, TILE_ROWS)])


@functools.lru_cache(maxsize=None)
def _make_segsum(d):
    mesh = plsc.VectorSubcoreMesh(core_axis_name="c", subcore_axis_name="s",
                                  num_cores=2, num_subcores=16)
    if d % 128 != 0:
        return pl.kernel(
            functools.partial(_segsum_deep_body, d=d),
            out_type=jax.ShapeDtypeStruct((2, N_PAD, d), jnp.float32),
            mesh=mesh,
            compiler_params=pltpu.CompilerParams(use_tc_tiling_on_sc=False),
            scratch_types=[
                pltpu.VMEM((max(KA, KB), CHUNK), jnp.int32),
                pltpu.VMEM((max(KA, KB), CHUNK), jnp.int32),
                pltpu.VMEM((NBUF, CHUNK, d), jnp.float32),
                pltpu.VMEM_SHARED((N_PAD, d), jnp.float32),
                pltpu.SemaphoreType.DMA,
                pltpu.SemaphoreType.DMA,
                pltpu.SemaphoreType.DMA,
                pltpu.SemaphoreType.DMA,
            ],
        )
    return pl.kernel(
        functools.partial(_segsum_body, d=d),
        out_type=jax.ShapeDtypeStruct((2, N_PAD, d), jnp.float32),
        mesh=mesh,
        scratch_types=[
            pltpu.VMEM((STAGE_MAX, CHUNK), jnp.int32),
            pltpu.VMEM((STAGE_MAX, CHUNK), jnp.int32),
            pltpu.VMEM((2, CHUNK, d), jnp.float32),
            pltpu.VMEM_SHARED((N_PAD, d), jnp.float32),
            pltpu.SemaphoreType.DMA,
            pltpu.SemaphoreType.DMA,
        ],
    )


def _pad2(a, shape):
    out = jnp.zeros(shape, jnp.float32)
    return out.at[:a.shape[0], :a.shape[1]].set(a)


def kernel(x, edge_index, W1l, b1, W1r, W2l, b2, W2r, Wml, bm, Wmr, Wvl, bv, Wvr):
    f32 = jnp.float32
    xp = jnp.zeros((N_PAD, 128), f32).at[:N_NODES].set(x)

    src = edge_index[0].astype(jnp.int32)
    dst = edge_index[1].astype(jnp.int32)
    pad_e = E_PAD - E_EDGES
    src_p = jnp.concatenate([src, jnp.zeros((pad_e,), jnp.int32)]
                            ).reshape(NW * K_CHUNKS, CHUNK)
    dst_p = jnp.concatenate([dst, jnp.full((pad_e,), N_PAD - 1, jnp.int32)]
                            ).reshape(NW * K_CHUNKS, CHUNK)

    w1l = _pad2(W1l.T, (128, D1))
    w1r = _pad2(W1r.T, (128, D1))
    b1p = _pad2(b1[None, :], (1, D1))
    w2l = _pad2(W2l.T, (D1, D2))
    w2r = _pad2(W2r.T, (D1, D2))
    b2p = _pad2(b2[None, :], (1, D2))
    wml = _pad2(Wml.T, (D2, OUT_D))
    wmr = _pad2(Wmr.T, (D2, OUT_D))
    bmp = _pad2(bm[None, :], (1, OUT_D))
    wvl = _pad2(Wvl.T, (D2, OUT_D))
    wvr = _pad2(Wvr.T, (D2, OUT_D))
    bvp = _pad2(bv[None, :], (1, OUT_D))

    t1, s1 = _mm2(xp, w1l, w1r, b1p, D1, DEG1)
    agg1 = _make_segsum(D1)(t1, src_p, dst_p)
    t2, s2 = _mid(agg1, s1, w2l, w2r, b2p, D1, D2, DEG1, DEG2)
    agg2 = _make_segsum(D2)(t2, src_p, dst_p)
    h2 = _h2(agg2, s2, D2, DEG2)
    m2 = _make_segsum(D2)(h2, src_p, dst_p)
    mu, lv = _heads(m2, h2, wml, wmr, bmp, wvl, wvr, bvp, D2, DEG2)
    return mu, lv


# speedup vs baseline: 1.0069x; 1.0069x over previous
import functools

import jax
import jax.numpy as jnp
from jax import lax
from jax.experimental import pallas as pl
from jax.experimental.pallas import tpu as pltpu
from jax.experimental.pallas import tpu_sc as plsc

N_NODES = 10000
N_PAD = 10240
E_EDGES = 320000
NW = 32
CHUNK = 128
K_CHUNKS = 80
E_PAD = NW * K_CHUNKS * CHUNK
TILE_ROWS = N_PAD // 16

BLK = 512
D1 = 128
D2 = 48
DEG1 = 117
DEG2 = 42
OUT_D = 24

_PREC = lax.Precision.DEFAULT
OUT_BLK = 400
OUT_GRID = N_NODES // OUT_BLK


def _mm2_body(x_ref, wl_ref, wr_ref, b_ref, t_ref, s_ref, *, ones_col):
    x = x_ref[...]
    t = jnp.dot(x, wl_ref[...], preferred_element_type=jnp.float32,
                precision=_PREC)
    col = lax.broadcasted_iota(jnp.int32, t.shape, 1)
    t_ref[...] = jnp.where(col == ones_col, 1.0, t)
    s_ref[...] = jnp.dot(x, wr_ref[...], preferred_element_type=jnp.float32,
                         precision=_PREC) + b_ref[...]


def _mid_body(a0_ref, a1_ref, s_ref, wl_ref, wr_ref, b_ref, t_ref, s2_ref,
              *, deg_col, ones_col):
    asum = a0_ref[0] + a1_ref[0]
    deg = asum[:, deg_col:deg_col + 1]
    inv = 1.0 / jnp.maximum(deg, 1.0)
    h = jnp.maximum(asum * inv + s_ref[...], 0.0)
    t = jnp.dot(h, wl_ref[...], preferred_element_type=jnp.float32,
                precision=_PREC)
    col = lax.broadcasted_iota(jnp.int32, t.shape, 1)
    t_ref[...] = jnp.where(col == ones_col, 1.0, t)
    s2_ref[...] = jnp.dot(h, wr_ref[...], preferred_element_type=jnp.float32,
                          precision=_PREC) + b_ref[...]


def _h2_body(a0_ref, a1_ref, s_ref, h_ref, *, deg_col):
    asum = a0_ref[0] + a1_ref[0]
    deg = asum[:, deg_col:deg_col + 1]
    inv = 1.0 / jnp.maximum(deg, 1.0)
    h = jnp.maximum(asum * inv + s_ref[...], 0.0)
    col = lax.broadcasted_iota(jnp.int32, h.shape, 1)
    h_ref[...] = jnp.where(col == deg_col, 1.0, h)


def _out_body(m0_ref, m1_ref, h_ref, wml_ref, wmr_ref, bm_ref,
              wvl_ref, wvr_ref, bv_ref, mu_ref, lv_ref, *, deg_col):
    msum = m0_ref[0] + m1_ref[0]
    deg = msum[:, deg_col:deg_col + 1]
    mean = msum * (1.0 / jnp.maximum(deg, 1.0))
    h = h_ref[...]
    mu_ref[...] = (jnp.dot(mean, wml_ref[...], preferred_element_type=jnp.float32, precision=_PREC)
                   + jnp.dot(h, wmr_ref[...], preferred_element_type=jnp.float32, precision=_PREC)
                   + bm_ref[...])
    lv_ref[...] = (jnp.dot(mean, wvl_ref[...], preferred_element_type=jnp.float32, precision=_PREC)
                   + jnp.dot(h, wvr_ref[...], preferred_element_type=jnp.float32, precision=_PREC)
                   + bv_ref[...])


def _row_blocks(d):
    return pl.BlockSpec((BLK, d), lambda i: (i, 0))


def _full(shape):
    ndim = len(shape)
    return pl.BlockSpec(shape, lambda i: (0,) * ndim)


def _agg_part(part, d):
    return pl.BlockSpec((1, BLK, d), lambda i, _p=part: (_p, i, 0))


_GRID = N_PAD // BLK


def _mm2(xp, wl, wr, b, d_out, ones_col):
    return pl.pallas_call(
        functools.partial(_mm2_body, ones_col=ones_col),
        grid=(_GRID,),
        in_specs=[_row_blocks(128), _full(wl.shape), _full(wr.shape), _full(b.shape)],
        out_specs=[_row_blocks(d_out), _row_blocks(d_out)],
        out_shape=[jax.ShapeDtypeStruct((N_PAD, d_out), jnp.float32)] * 2,
    )(xp, wl, wr, b)


def _mid(agg, s1, wl, wr, b, d_in, d_out, deg_col, ones_col):
    return pl.pallas_call(
        functools.partial(_mid_body, deg_col=deg_col, ones_col=ones_col),
        grid=(_GRID,),
        in_specs=[_agg_part(0, d_in), _agg_part(1, d_in), _row_blocks(d_in),
                  _full(wl.shape), _full(wr.shape), _full(b.shape)],
        out_specs=[_row_blocks(d_out), _row_blocks(d_out)],
        out_shape=[jax.ShapeDtypeStruct((N_PAD, d_out), jnp.float32)] * 2,
    )(agg, agg, s1, wl, wr, b)


def _h2(agg, s2, d, deg_col):
    return pl.pallas_call(
        functools.partial(_h2_body, deg_col=deg_col),
        grid=(_GRID,),
        in_specs=[_agg_part(0, d), _agg_part(1, d), _row_blocks(d)],
        out_specs=_row_blocks(d),
        out_shape=jax.ShapeDtypeStruct((N_PAD, d), jnp.float32),
    )(agg, agg, s2)


def _heads(agg, h2, wml, wmr, bm, wvl, wvr, bv, d, deg_col):
    rb = pl.BlockSpec((OUT_BLK, d), lambda i: (i, 0))
    ab0 = pl.BlockSpec((1, OUT_BLK, d), lambda i: (0, i, 0))
    ab1 = pl.BlockSpec((1, OUT_BLK, d), lambda i: (1, i, 0))
    ob = pl.BlockSpec((OUT_BLK, OUT_D), lambda i: (i, 0))
    return pl.pallas_call(
        functools.partial(_out_body, deg_col=deg_col),
        grid=(OUT_GRID,),
        in_specs=[ab0, ab1, rb,
                  _full(wml.shape), _full(wmr.shape), _full(bm.shape),
                  _full(wvl.shape), _full(wvr.shape), _full(bv.shape)],
        out_specs=[ob, ob],
        out_shape=[jax.ShapeDtypeStruct((N_NODES, OUT_D), jnp.float32)] * 2,
    )(agg, agg, h2, wml, wmr, bm, wvl, wvr, bv)


KA = 128
KB = K_CHUNKS * 2 - KA
N_STAGES = 4
STAGE_MAX = max(KA, KB) // N_STAGES


def _segsum_body(t_hbm, src_hbm, dst_hbm, out_hbm,
                 src_v, dst_v, rows_v, acc, sem0, sem1, *, d):
    c = lax.axis_index("c")
    s = lax.axis_index("s")

    zv = jnp.zeros((16,), jnp.float32)

    def zrow(i, carry):
        for k2 in range(d // 16):
            rows_v[0, i, pl.ds(k2 * 16, 16)] = zv
        return carry

    lax.fori_loop(0, CHUNK, zrow, 0)
    base = s * TILE_ROWS
    for r in range(TILE_ROWS // CHUNK):
        pltpu.sync_copy(rows_v.at[0], acc.at[pl.ds(base + r * CHUNK, CHUNK)])
    plsc.subcore_barrier()

    def gather(j, buf, sem):
        pltpu.async_copy(t_hbm.at[src_v.at[j]], rows_v.at[buf], sem)

    def gwait(j, buf, sem):
        pltpu.make_async_copy(t_hbm.at[src_v.at[j]], rows_v.at[buf], sem).wait()

    def run(chunk_base, k):
        stage = k // N_STAGES
        pairs = stage // 2
        for h in range(N_STAGES):
            off = chunk_base + h * stage
            pltpu.sync_copy(src_hbm.at[pl.ds(off, stage)],
                            src_v.at[pl.ds(0, stage)])
            pltpu.sync_copy(dst_hbm.at[pl.ds(off, stage)],
                            dst_v.at[pl.ds(0, stage)])
            gather(0, 0, sem0)

            def body(jj, carry):
                j0 = 2 * jj
                gwait(j0, 0, sem0)
                gather(j0 + 1, 1, sem1)
                pltpu.sync_copy(rows_v.at[0], acc.at[dst_v.at[j0]], add=True)

                @pl.when(jj < pairs - 1)
                def _():
                    gather(j0 + 2, 0, sem0)

                gwait(j0 + 1, 1, sem1)
                pltpu.sync_copy(rows_v.at[1], acc.at[dst_v.at[j0 + 1]],
                                add=True)
                return carry

            lax.fori_loop(0, pairs, body, 0)

    @pl.when(c == 0)
    def _():
        run(s * KA, KA)

    if KB > 0:
        @pl.when(c == 1)
        def _():
            run(16 * KA + s * KB, KB)

    plsc.subcore_barrier()
    pltpu.sync_copy(acc.at[pl.ds(base, TILE_ROWS)],
                    out_hbm.at[c, pl.ds(base, TILE_ROWS)])


@functools.lru_cache(maxsize=None)
def _make_segsum(d):
    mesh = plsc.VectorSubcoreMesh(core_axis_name="c", subcore_axis_name="s",
                                  num_cores=2, num_subcores=16)
    params = None
    if d % 128 != 0:
        params = pltpu.CompilerParams(use_tc_tiling_on_sc=False)
    return pl.kernel(
        functools.partial(_segsum_body, d=d),
        out_type=jax.ShapeDtypeStruct((2, N_PAD, d), jnp.float32),
        mesh=mesh,
        compiler_params=params,
        scratch_types=[
            pltpu.VMEM((STAGE_MAX, CHUNK), jnp.int32),
            pltpu.VMEM((STAGE_MAX, CHUNK), jnp.int32),
            pltpu.VMEM((2, CHUNK, d), jnp.float32),
            pltpu.VMEM_SHARED((N_PAD, d), jnp.float32),
            pltpu.SemaphoreType.DMA,
            pltpu.SemaphoreType.DMA,
        ],
    )


def _pad2(a, shape):
    out = jnp.zeros(shape, jnp.float32)
    return out.at[:a.shape[0], :a.shape[1]].set(a)


def kernel(x, edge_index, W1l, b1, W1r, W2l, b2, W2r, Wml, bm, Wmr, Wvl, bv, Wvr):
    f32 = jnp.float32
    xp = jnp.zeros((N_PAD, 128), f32).at[:N_NODES].set(x)

    src = edge_index[0].astype(jnp.int32)
    dst = edge_index[1].astype(jnp.int32)
    pad_e = E_PAD - E_EDGES
    src_p = jnp.concatenate([src, jnp.zeros((pad_e,), jnp.int32)]
                            ).reshape(NW * K_CHUNKS, CHUNK)
    dst_p = jnp.concatenate([dst, jnp.full((pad_e,), N_PAD - 1, jnp.int32)]
                            ).reshape(NW * K_CHUNKS, CHUNK)

    w1l = _pad2(W1l.T, (128, D1))
    w1r = _pad2(W1r.T, (128, D1))
    b1p = _pad2(b1[None, :], (1, D1))
    w2l = _pad2(W2l.T, (D1, D2))
    w2r = _pad2(W2r.T, (D1, D2))
    b2p = _pad2(b2[None, :], (1, D2))
    wml = _pad2(Wml.T, (D2, OUT_D))
    wmr = _pad2(Wmr.T, (D2, OUT_D))
    bmp = _pad2(bm[None, :], (1, OUT_D))
    wvl = _pad2(Wvl.T, (D2, OUT_D))
    wvr = _pad2(Wvr.T, (D2, OUT_D))
    bvp = _pad2(bv[None, :], (1, OUT_D))

    t1, s1 = _mm2(xp, w1l, w1r, b1p, D1, DEG1)
    agg1 = _make_segsum(D1)(t1, src_p, dst_p)
    t2, s2 = _mid(agg1, s1, w2l, w2r, b2p, D1, D2, DEG1, DEG2)
    agg2 = _make_segsum(D2)(t2, src_p, dst_p)
    h2 = _h2(agg2, s2, D2, DEG2)
    m2 = _make_segsum(D2)(h2, src_p, dst_p)
    mu, lv = _heads(m2, h2, wml, wmr, bmp, wvl, wvr, bvp, D2, DEG2)
    return mu, lv
